# Initial kernel scaffold; baseline (speedup 1.0000x reference)
import jax
import jax.numpy as jnp
from jax.experimental import pallas as pl


def kernel(h, edge_belongs, edge_transfers, Ws, bs):
    raise NotImplementedError("write your pallas kernel here")


# speedup vs baseline: 2.4864x; 2.4864x over previous
import functools

import jax
import jax.numpy as jnp
from jax import lax
from jax.experimental import pallas as pl
from jax.experimental.pallas import tpu as pltpu
from jax.experimental.pallas import tpu_sc as plsc

N = 10000
D = 128
E = 320000

NC = 2
NS = 16
G = 128
GROUPS = -(-((E + NS * G - 1) // (NS * G)) // 8) * 8
E_TILE = GROUPS * G
E_PAD = NS * E_TILE
ACC = N + 112
ROWS_T = ACC // NS
NBUF = 2
IBLK = 40
NIB = GROUPS // IBLK


def _agg_body(x_hbm, srcs_hbm, dsts_hbm, zeros_hbm, out_hbm,
              acc, sidx, didx, rows, gsem, ssem):
    c = lax.axis_index("c")
    s = lax.axis_index("s")

    base_r = s * ROWS_T
    pltpu.sync_copy(zeros_hbm, acc.at[pl.ds(base_r, ROWS_T)])
    plsc.subcore_barrier()

    tb = s * GROUPS

    def start_gather(j, b):
        pltpu.async_copy(x_hbm.at[sidx.at[j]], rows.at[b], gsem)

    def wait_gather():
        pltpu.make_async_copy(x_hbm.at[pl.ds(0, G)], rows.at[0], gsem).wait()

    def start_scatter(j, b):
        pltpu.async_copy(rows.at[b], acc.at[didx.at[j]], ssem, add=True)

    def wait_scatter():
        pltpu.make_async_copy(x_hbm.at[pl.ds(0, G)], rows.at[0], ssem).wait()

    def block(ib, carry):
        boff = tb + ib * IBLK
        pltpu.sync_copy(srcs_hbm.at[c, pl.ds(boff, IBLK)], sidx)
        pltpu.sync_copy(dsts_hbm.at[c, pl.ds(boff, IBLK)], didx)
        start_gather(0, 0)

        def body(j, carry2):
            wait_gather()

            @pl.when(j > 0)
            def _():
                wait_scatter()

            @pl.when(j + 1 < IBLK)
            def _():
                start_gather(j + 1, lax.rem(j + 1, 2))

            start_scatter(j, lax.rem(j, 2))
            return carry2

        lax.fori_loop(0, IBLK, body, 0)
        wait_scatter()
        return carry

    lax.fori_loop(0, NIB, block, 0)
    plsc.subcore_barrier()

    pltpu.sync_copy(acc.at[pl.ds(base_r, ROWS_T)], out_hbm.at[c, pl.ds(base_r, ROWS_T)])


_agg_call = pl.kernel(
    _agg_body,
    out_type=jax.ShapeDtypeStruct((NC, ACC, D), jnp.float32),
    mesh=plsc.VectorSubcoreMesh(core_axis_name="c", subcore_axis_name="s"),
    scratch_types=[
        pltpu.VMEM_SHARED((ACC, D), jnp.float32),
        pltpu.VMEM((IBLK, G), jnp.int32),
        pltpu.VMEM((IBLK, G), jnp.int32),
        pltpu.VMEM((NBUF, G, D), jnp.float32),
        pltpu.SemaphoreType.DMA,
        pltpu.SemaphoreType.DMA,
    ],
)


BLK = 1000


def _tc_layer_body(x_ref, ss_ref, deg_ref, W_ref, b_ref, o_ref):
    xb = x_ref[...]
    W = W_ref[...]
    b = b_ref[...]
    acc = None
    for r in range(2):
        deg = deg_ref[r, :, 0:1]
        inv = 1.0 / jnp.maximum(deg, 1.0)
        ss = ss_ref[r]
        o = (
            jnp.dot(xb, W[r, 0], preferred_element_type=jnp.float32,
                    precision=lax.Precision.HIGHEST)
            + b[r, 0]
            + jnp.dot(ss * inv, W[r, 1], preferred_element_type=jnp.float32,
                      precision=lax.Precision.HIGHEST)
            + b[r, 1]
        )
        o = jnp.maximum(o, 0.0)
        acc = o if acc is None else acc + o
    o_ref[...] = acc * 0.5


def _tc_layer(x, ss, deg, W, b):
    return pl.pallas_call(
        _tc_layer_body,
        grid=(N // BLK,),
        in_specs=[
            pl.BlockSpec((BLK, D), lambda i: (i, 0)),
            pl.BlockSpec((NC, BLK, D), lambda i: (0, i, 0)),
            pl.BlockSpec((NC, BLK, D), lambda i: (0, i, 0)),
            pl.BlockSpec((NC, NC, D, D), lambda i: (0, 0, 0, 0)),
            pl.BlockSpec((NC, NC, D), lambda i: (0, 0, 0)),
        ],
        out_specs=pl.BlockSpec((BLK, D), lambda i: (i, 0)),
        out_shape=jax.ShapeDtypeStruct((N, D), jnp.float32),
    )(x, ss, deg, W, b)


def _pad_stack(eb, et):
    pad = E_PAD - E
    srcs = jnp.stack([
        jnp.concatenate([eb[0], jnp.zeros((pad,), jnp.int32)]),
        jnp.concatenate([et[0], jnp.zeros((pad,), jnp.int32)]),
    ]).reshape(NC, NS * GROUPS, G)
    dsts = jnp.stack([
        jnp.concatenate([eb[1], jnp.full((pad,), N, jnp.int32)]),
        jnp.concatenate([et[1], jnp.full((pad,), N, jnp.int32)]),
    ]).reshape(NC, NS * GROUPS, G)
    return srcs, dsts


def kernel(h, edge_belongs, edge_transfers, Ws, bs):
    srcs, dsts = _pad_stack(edge_belongs, edge_transfers)
    zeros_d = jnp.zeros((ROWS_T, D), jnp.float32)
    ones_nd = jnp.ones((N, D), jnp.float32)
    deg = _agg_call(ones_nd, srcs, dsts, zeros_d)
    x = h
    for l in range(Ws.shape[0]):
        ss = _agg_call(x, srcs, dsts, zeros_d)
        x = _tc_layer(x, ss, deg, Ws[l], bs[l])
    return x


# speedup vs baseline: 3.1011x; 1.2472x over previous
import functools

import jax
import jax.numpy as jnp
from jax import lax
from jax.experimental import pallas as pl
from jax.experimental.pallas import tpu as pltpu
from jax.experimental.pallas import tpu_sc as plsc

N = 10000
D = 128
E = 320000

NC = 2
NS = 16
G = 128
GROUPS = -(-((E + NS * G - 1) // (NS * G)) // 8) * 8
E_TILE = GROUPS * G
E_PAD = NS * E_TILE
ACC = N + 112
ROWS_T = ACC // NS
NBUF = 2
IBLK = 40
NIB = GROUPS // IBLK


def _agg_body(x_hbm, srcs_hbm, dsts_hbm, zeros_hbm, out_hbm,
              acc, sidx, didx, rows, gsem, ssem):
    c = lax.axis_index("c")
    s = lax.axis_index("s")

    base_r = s * ROWS_T
    pltpu.sync_copy(zeros_hbm, acc.at[pl.ds(base_r, ROWS_T)])
    plsc.subcore_barrier()

    tb = s * GROUPS

    def start_gather(j, b):
        pltpu.async_copy(x_hbm.at[sidx.at[j]], rows.at[b], gsem)

    def wait_gather():
        pltpu.make_async_copy(x_hbm.at[pl.ds(0, G)], rows.at[0], gsem).wait()

    def start_scatter(j, b):
        pltpu.async_copy(rows.at[b], acc.at[didx.at[j]], ssem, add=True)

    def wait_scatter():
        pltpu.make_async_copy(x_hbm.at[pl.ds(0, G)], rows.at[0], ssem).wait()

    def block(ib, carry):
        boff = tb + ib * IBLK
        pltpu.sync_copy(srcs_hbm.at[c, pl.ds(boff, IBLK)], sidx)
        pltpu.sync_copy(dsts_hbm.at[c, pl.ds(boff, IBLK)], didx)
        start_gather(0, 0)

        def body(j, carry2):
            wait_gather()

            @pl.when(j > 0)
            def _():
                wait_scatter()

            @pl.when(j + 1 < IBLK)
            def _():
                start_gather(j + 1, lax.rem(j + 1, 2))

            start_scatter(j, lax.rem(j, 2))
            return carry2

        lax.fori_loop(0, IBLK, body, 0)
        wait_scatter()
        return carry

    lax.fori_loop(0, NIB, block, 0)
    plsc.subcore_barrier()

    pltpu.sync_copy(acc.at[pl.ds(base_r, ROWS_T)], out_hbm.at[c, pl.ds(base_r, ROWS_T)])


_agg_call = pl.kernel(
    _agg_body,
    out_type=jax.ShapeDtypeStruct((NC, ACC, D), jnp.float32),
    mesh=plsc.VectorSubcoreMesh(core_axis_name="c", subcore_axis_name="s"),
    scratch_types=[
        pltpu.VMEM_SHARED((ACC, D), jnp.float32),
        pltpu.VMEM((IBLK, G), jnp.int32),
        pltpu.VMEM((IBLK, G), jnp.int32),
        pltpu.VMEM((NBUF, G, D), jnp.float32),
        pltpu.SemaphoreType.DMA,
        pltpu.SemaphoreType.DMA,
    ],
)


SLAG = 8


def _deg_body(dsts_hbm, ones_hbm, zeros_hbm, out_hbm, acc, didx, ones_buf, ssem):
    c = lax.axis_index("c")
    s = lax.axis_index("s")
    base_r = s * ROWS_T
    pltpu.sync_copy(zeros_hbm, acc.at[pl.ds(base_r, ROWS_T)])
    pltpu.sync_copy(ones_hbm, ones_buf)
    plsc.subcore_barrier()

    tb = s * GROUPS

    def wait_scatter():
        pltpu.make_async_copy(ones_buf, acc.at[pl.ds(0, G)], ssem).wait()

    def block(ib, carry):
        boff = tb + ib * IBLK
        pltpu.sync_copy(dsts_hbm.at[c, pl.ds(boff, IBLK)], didx)

        def body(j, c2):
            pltpu.async_copy(ones_buf, acc.at[didx.at[j]], ssem, add=True)

            @pl.when(j >= SLAG)
            def _():
                wait_scatter()

            return c2

        lax.fori_loop(0, IBLK, body, 0)
        lax.fori_loop(0, SLAG, lambda j, c2: (wait_scatter(), c2)[1], 0)
        return carry

    lax.fori_loop(0, NIB, block, 0)
    plsc.subcore_barrier()
    pltpu.sync_copy(acc.at[pl.ds(base_r, ROWS_T)], out_hbm.at[c, pl.ds(base_r, ROWS_T)])


_deg_call = pl.kernel(
    _deg_body,
    out_type=jax.ShapeDtypeStruct((NC, ACC, D), jnp.float32),
    mesh=plsc.VectorSubcoreMesh(core_axis_name="c", subcore_axis_name="s"),
    scratch_types=[
        pltpu.VMEM_SHARED((ACC, D), jnp.float32),
        pltpu.VMEM((IBLK, G), jnp.int32),
        pltpu.VMEM((G, D), jnp.float32),
        pltpu.SemaphoreType.DMA,
    ],
)


BLK = 1000


def _tc_layer_body(x_ref, ss_ref, deg_ref, W_ref, b_ref, o_ref):
    xb = x_ref[...]
    W = W_ref[...]
    b = b_ref[...]
    acc = None
    for r in range(2):
        deg = deg_ref[r, :, 0:1]
        inv = 1.0 / jnp.maximum(deg, 1.0)
        ss = ss_ref[r]
        o = (
            jnp.dot(xb, W[r, 0], preferred_element_type=jnp.float32,
                    precision=lax.Precision.HIGHEST)
            + b[r, 0]
            + jnp.dot(ss * inv, W[r, 1], preferred_element_type=jnp.float32,
                      precision=lax.Precision.HIGHEST)
            + b[r, 1]
        )
        o = jnp.maximum(o, 0.0)
        acc = o if acc is None else acc + o
    o_ref[...] = acc * 0.5


def _tc_layer(x, ss, deg, W, b):
    return pl.pallas_call(
        _tc_layer_body,
        grid=(N // BLK,),
        in_specs=[
            pl.BlockSpec((BLK, D), lambda i: (i, 0)),
            pl.BlockSpec((NC, BLK, D), lambda i: (0, i, 0)),
            pl.BlockSpec((NC, BLK, D), lambda i: (0, i, 0)),
            pl.BlockSpec((NC, NC, D, D), lambda i: (0, 0, 0, 0)),
            pl.BlockSpec((NC, NC, D), lambda i: (0, 0, 0)),
        ],
        out_specs=pl.BlockSpec((BLK, D), lambda i: (i, 0)),
        out_shape=jax.ShapeDtypeStruct((N, D), jnp.float32),
    )(x, ss, deg, W, b)


def _pad_stack(eb, et):
    pad = E_PAD - E
    srcs = jnp.stack([
        jnp.concatenate([eb[0], jnp.zeros((pad,), jnp.int32)]),
        jnp.concatenate([et[0], jnp.zeros((pad,), jnp.int32)]),
    ]).reshape(NC, NS * GROUPS, G)
    dsts = jnp.stack([
        jnp.concatenate([eb[1], jnp.full((pad,), N, jnp.int32)]),
        jnp.concatenate([et[1], jnp.full((pad,), N, jnp.int32)]),
    ]).reshape(NC, NS * GROUPS, G)
    return srcs, dsts


def kernel(h, edge_belongs, edge_transfers, Ws, bs):
    srcs, dsts = _pad_stack(edge_belongs, edge_transfers)
    zeros_d = jnp.zeros((ROWS_T, D), jnp.float32)
    ones_gd = jnp.ones((G, D), jnp.float32)
    deg = _deg_call(dsts, ones_gd, zeros_d)
    x = h
    for l in range(Ws.shape[0]):
        ss = _agg_call(x, srcs, dsts, zeros_d)
        x = _tc_layer(x, ss, deg, Ws[l], bs[l])
    return x
